# baseline (speedup 1.0000x reference)
import functools
import math

import jax
import jax.numpy as jnp
from jax import lax
from jax.experimental import pallas as pl
from jax.experimental.pallas import tpu as pltpu
from jax.experimental.pallas import tpu_sc as plsc

_PRED_CLIP = 0.001
_LO = math.log(_PRED_CLIP / (1.0 - _PRED_CLIP))
_HI = math.log((1.0 - _PRED_CLIP) / _PRED_CLIP)
_BB = 256
_B = 1024
_NCHUNK = 2
_CB = _B // _NCHUNK
_NW = 32
_SPW = _CB // _NW


def _prep_layer(cm, cb, w, S, shift, P):
    s = cm.shape[1]
    pf, pb = shift, S - s - shift
    cmt = jnp.pad(jnp.transpose(cm[0], (1, 0, 2)), ((0, 0), (pf, pb), (0, 0)))
    cmT = jnp.transpose(cmt.reshape(4 * S, cm.shape[3]), (1, 0))
    cbt = jnp.pad(jnp.transpose(cb[0, :, :, 0], (1, 0)), ((0, 0), (pf, pb)),
                  constant_values=jnp.inf)
    cbp = cbt.reshape(1, 4 * S)
    wp = jnp.pad(jnp.transpose(w[0], (1, 0, 2)),
                 ((0, 0), (pf, pb), (0, P - w.shape[3])))
    wT = jnp.transpose(wp.reshape(16 * S, P), (1, 0))
    return cmT, cbp, wT


def _ctx_idx(x, cmT, cbp, S):
    d = jnp.dot(x, cmT, preferred_element_type=jnp.float32)
    bits = (d > cbp).astype(jnp.float32)
    return (bits[:, 0 * S:1 * S] + 2.0 * bits[:, 1 * S:2 * S]
            + 4.0 * bits[:, 2 * S:3 * S] + 8.0 * bits[:, 3 * S:4 * S])


def _select16(a, idx, S):
    out = jnp.where(idx == 0.0, a[:, 0:S], 0.0)
    for k in range(1, 16):
        out = out + jnp.where(idx == float(k), a[:, k * S:(k + 1) * S], 0.0)
    return out


def _tc_a_body(x_ref, sc_ref, cm0_ref, cb0_ref, w0_ref, cm1_ref, cb1_ref,
               cm2_ref, cb2_ref, a0_ref, off0_ref, off1_ref, idx2_ref):
    x = x_ref[...]
    lane256 = lax.broadcasted_iota(jnp.int32, (1, 256), 1)
    lane128 = lax.broadcasted_iota(jnp.int32, (1, 128), 1).astype(jnp.float32)

    xc = jnp.clip(x, _PRED_CLIP, 1.0 - _PRED_CLIP)
    l0 = jnp.log(xc / (1.0 - xc))
    l0 = jnp.where(lane256 == 0, sc_ref[0], l0)

    idx0 = _ctx_idx(x, cm0_ref[...], cb0_ref[...], 128)
    off0_ref[...] = (idx0 * 128.0 + lane128).astype(jnp.int32)
    idx1 = _ctx_idx(x, cm1_ref[...], cb1_ref[...], 128)
    off1_ref[...] = (idx1 * 128.0 + lane128).astype(jnp.int32)
    idx2_ref[...] = _ctx_idx(x, cm2_ref[...], cb2_ref[...], 8)
    a0_ref[...] = jnp.dot(l0, w0_ref[...], preferred_element_type=jnp.float32)


def _sc_sel_body(a_hbm, off_hbm, out_hbm, a_v, off_v, out_v):
    cid = lax.axis_index("c")
    sid = lax.axis_index("s")
    wid = sid * 2 + cid
    base = wid * _SPW
    pltpu.sync_copy(a_hbm.at[pl.ds(base, _SPW)], a_v)
    pltpu.sync_copy(off_hbm.at[pl.ds(base, _SPW)], off_v)
    iota16 = lax.iota(jnp.int32, 16)

    def jbody(j, carry):
        jv = jnp.full((16,), j, jnp.int32)
        for g in range(8):
            col = g * 16 + iota16
            off = plsc.load_gather(off_v, [jv, col])
            val = plsc.load_gather(a_v, [jv, off])
            plsc.store_scatter(out_v, [jv, col], val)
        return carry

    lax.fori_loop(0, _SPW, jbody, 0)
    pltpu.sync_copy(out_v, out_hbm.at[pl.ds(base, _SPW)])


@functools.cache
def _get_sc_select():
    return pl.kernel(
        _sc_sel_body,
        out_type=jax.ShapeDtypeStruct((_CB, 128), jnp.float32),
        mesh=plsc.VectorSubcoreMesh(core_axis_name="c", subcore_axis_name="s"),
        compiler_params=pltpu.CompilerParams(needs_layout_passes=False),
        scratch_types=[
            pltpu.VMEM((_SPW, 2048), jnp.float32),
            pltpu.VMEM((_SPW, 128), jnp.int32),
            pltpu.VMEM((_SPW, 128), jnp.float32),
        ],
    )


def _tc_b_body(sel_ref, sc_ref, w_ref, a_ref, *, bias_slot):
    lane128 = lax.broadcasted_iota(jnp.int32, (1, 128), 1)
    l = jnp.where(lane128 == 0, sc_ref[bias_slot],
                  jnp.clip(sel_ref[...], _LO, _HI))
    a_ref[...] = jnp.dot(l, w_ref[...], preferred_element_type=jnp.float32)


def _tc_c_body(sel_ref, sc_ref, w2_ref, idx2_ref, o_ref):
    lane128 = lax.broadcasted_iota(jnp.int32, (1, 128), 1)
    l2 = jnp.where(lane128 == 0, sc_ref[2],
                   jnp.clip(sel_ref[...], _LO, _HI))
    a2 = jnp.dot(l2, w2_ref[...], preferred_element_type=jnp.float32)
    out2 = _select16(a2, idx2_ref[...], 8)
    o_ref[...] = jax.nn.sigmoid(jnp.clip(out2[:, 0:1], _LO, _HI))


def kernel(x, base_bias, bias_0, bias_1, ctx_maps_0, ctx_bias_0, weights_0,
           ctx_maps_1, ctx_bias_1, weights_1, ctx_maps_2, ctx_bias_2,
           weights_2):
    cm0T, cb0, w0T = _prep_layer(ctx_maps_0, ctx_bias_0, weights_0, 128, 1, 256)
    cm1T, cb1, w1T = _prep_layer(ctx_maps_1, ctx_bias_1, weights_1, 128, 1, 128)
    cm2T, cb2, w2T = _prep_layer(ctx_maps_2, ctx_bias_2, weights_2, 8, 0, 128)
    scalars = jnp.stack([base_bias, bias_0[0, 0, 0], bias_1[0, 0, 0]])

    rep = lambda i: (0, 0)
    blk = lambda i: (i, 0)
    grid = (_CB // _BB,)

    def chain(xc):
        a0, off0, off1, idx2 = pl.pallas_call(
            _tc_a_body,
            grid=grid,
            in_specs=[
                pl.BlockSpec((_BB, 256), blk),
                pl.BlockSpec(memory_space=pltpu.SMEM),
                pl.BlockSpec((256, 512), rep),
                pl.BlockSpec((1, 512), rep),
                pl.BlockSpec((256, 2048), rep),
                pl.BlockSpec((256, 512), rep),
                pl.BlockSpec((1, 512), rep),
                pl.BlockSpec((256, 32), rep),
                pl.BlockSpec((1, 32), rep),
            ],
            out_specs=[
                pl.BlockSpec((_BB, 2048), blk),
                pl.BlockSpec((_BB, 128), blk),
                pl.BlockSpec((_BB, 128), blk),
                pl.BlockSpec((_BB, 8), blk),
            ],
            out_shape=[
                jax.ShapeDtypeStruct((_CB, 2048), jnp.float32),
                jax.ShapeDtypeStruct((_CB, 128), jnp.int32),
                jax.ShapeDtypeStruct((_CB, 128), jnp.int32),
                jax.ShapeDtypeStruct((_CB, 8), jnp.float32),
            ],
        )(xc, scalars, cm0T, cb0, w0T, cm1T, cb1, cm2T, cb2)

        sel0 = _get_sc_select()(a0, off0)

        a1 = pl.pallas_call(
            functools.partial(_tc_b_body, bias_slot=1),
            grid=grid,
            in_specs=[
                pl.BlockSpec((_BB, 128), blk),
                pl.BlockSpec(memory_space=pltpu.SMEM),
                pl.BlockSpec((128, 2048), rep),
            ],
            out_specs=pl.BlockSpec((_BB, 2048), blk),
            out_shape=jax.ShapeDtypeStruct((_CB, 2048), jnp.float32),
        )(sel0, scalars, w1T)

        sel1 = _get_sc_select()(a1, off1)

        return pl.pallas_call(
            _tc_c_body,
            grid=grid,
            in_specs=[
                pl.BlockSpec((_BB, 128), blk),
                pl.BlockSpec(memory_space=pltpu.SMEM),
                pl.BlockSpec((128, 128), rep),
                pl.BlockSpec((_BB, 8), blk),
            ],
            out_specs=pl.BlockSpec((_BB, 1), blk),
            out_shape=jax.ShapeDtypeStruct((_CB, 1), jnp.float32),
        )(sel1, scalars, w2T, idx2)

    outs = [chain(x[c * _CB:(c + 1) * _CB]) for c in range(_NCHUNK)]
    return jnp.concatenate(outs, axis=0)


# speedup vs baseline: 1.0846x; 1.0846x over previous
import functools
import math

import jax
import jax.numpy as jnp
from jax import lax
from jax.experimental import pallas as pl
from jax.experimental.pallas import tpu as pltpu
from jax.experimental.pallas import tpu_sc as plsc

_PRED_CLIP = 0.001
_LO = math.log(_PRED_CLIP / (1.0 - _PRED_CLIP))
_HI = math.log((1.0 - _PRED_CLIP) / _PRED_CLIP)
_BB = 256
_B = 1024
_NCHUNK = 1
_CB = _B // _NCHUNK
_NW = 32
_SPW = _CB // _NW


def _prep_layer(cm, cb, w, S, shift, P):
    s = cm.shape[1]
    pf, pb = shift, S - s - shift
    cmt = jnp.pad(jnp.transpose(cm[0], (1, 0, 2)), ((0, 0), (pf, pb), (0, 0)))
    cmT = jnp.transpose(cmt.reshape(4 * S, cm.shape[3]), (1, 0))
    cbt = jnp.pad(jnp.transpose(cb[0, :, :, 0], (1, 0)), ((0, 0), (pf, pb)),
                  constant_values=jnp.inf)
    cbp = cbt.reshape(1, 4 * S)
    wp = jnp.pad(jnp.transpose(w[0], (1, 0, 2)),
                 ((0, 0), (pf, pb), (0, P - w.shape[3])))
    wT = jnp.transpose(wp.reshape(16 * S, P), (1, 0))
    return cmT, cbp, wT


def _ctx_idx(x, cmT, cbp, S):
    d = jnp.dot(x, cmT, preferred_element_type=jnp.float32)
    bits = (d > cbp).astype(jnp.float32)
    return (bits[:, 0 * S:1 * S] + 2.0 * bits[:, 1 * S:2 * S]
            + 4.0 * bits[:, 2 * S:3 * S] + 8.0 * bits[:, 3 * S:4 * S])


def _select16(a, idx, S):
    out = jnp.where(idx == 0.0, a[:, 0:S], 0.0)
    for k in range(1, 16):
        out = out + jnp.where(idx == float(k), a[:, k * S:(k + 1) * S], 0.0)
    return out


def _tc_a_body(x_ref, sc_ref, cm0_ref, cb0_ref, w0_ref, cm1_ref, cb1_ref,
               cm2_ref, cb2_ref, a0_ref, off0_ref, off1_ref, idx2_ref):
    x = x_ref[...]
    lane256 = lax.broadcasted_iota(jnp.int32, (1, 256), 1)
    lane128 = lax.broadcasted_iota(jnp.int32, (1, 128), 1).astype(jnp.float32)

    xc = jnp.clip(x, _PRED_CLIP, 1.0 - _PRED_CLIP)
    l0 = jnp.log(xc / (1.0 - xc))
    l0 = jnp.where(lane256 == 0, sc_ref[0], l0)

    idx0 = _ctx_idx(x, cm0_ref[...], cb0_ref[...], 128)
    off0_ref[...] = (idx0 * 128.0 + lane128).astype(jnp.int32)
    idx1 = _ctx_idx(x, cm1_ref[...], cb1_ref[...], 128)
    off1_ref[...] = (idx1 * 128.0 + lane128).astype(jnp.int32)
    idx2_ref[...] = _ctx_idx(x, cm2_ref[...], cb2_ref[...], 8)
    a0_ref[...] = jnp.dot(l0, w0_ref[...], preferred_element_type=jnp.float32)


def _sc_sel_body(a_hbm, off_hbm, out_hbm, a_v, off_v, out_v):
    cid = lax.axis_index("c")
    sid = lax.axis_index("s")
    wid = sid * 2 + cid
    base = wid * _SPW
    pltpu.sync_copy(a_hbm.at[pl.ds(base, _SPW)], a_v)
    pltpu.sync_copy(off_hbm.at[pl.ds(base, _SPW)], off_v)
    iota16 = lax.iota(jnp.int32, 16)

    def jbody(j, carry):
        jv = jnp.full((16,), j, jnp.int32)
        for g in range(8):
            off = off_v[j, pl.ds(g * 16, 16)]
            val = plsc.load_gather(a_v, [jv, off])
            out_v[j, pl.ds(g * 16, 16)] = val
        return carry

    lax.fori_loop(0, _SPW, jbody, 0)
    pltpu.sync_copy(out_v, out_hbm.at[pl.ds(base, _SPW)])


@functools.cache
def _get_sc_select():
    return pl.kernel(
        _sc_sel_body,
        out_type=jax.ShapeDtypeStruct((_CB, 128), jnp.float32),
        mesh=plsc.VectorSubcoreMesh(core_axis_name="c", subcore_axis_name="s"),
        compiler_params=pltpu.CompilerParams(needs_layout_passes=False),
        scratch_types=[
            pltpu.VMEM((_SPW, 2048), jnp.float32),
            pltpu.VMEM((_SPW, 128), jnp.int32),
            pltpu.VMEM((_SPW, 128), jnp.float32),
        ],
    )


def _tc_b_body(sel_ref, sc_ref, w_ref, a_ref, *, bias_slot):
    lane128 = lax.broadcasted_iota(jnp.int32, (1, 128), 1)
    l = jnp.where(lane128 == 0, sc_ref[bias_slot],
                  jnp.clip(sel_ref[...], _LO, _HI))
    a_ref[...] = jnp.dot(l, w_ref[...], preferred_element_type=jnp.float32)


def _tc_c_body(sel_ref, sc_ref, w2_ref, idx2_ref, o_ref):
    lane128 = lax.broadcasted_iota(jnp.int32, (1, 128), 1)
    l2 = jnp.where(lane128 == 0, sc_ref[2],
                   jnp.clip(sel_ref[...], _LO, _HI))
    a2 = jnp.dot(l2, w2_ref[...], preferred_element_type=jnp.float32)
    out2 = _select16(a2, idx2_ref[...], 8)
    o_ref[...] = jax.nn.sigmoid(jnp.clip(out2[:, 0:1], _LO, _HI))


def kernel(x, base_bias, bias_0, bias_1, ctx_maps_0, ctx_bias_0, weights_0,
           ctx_maps_1, ctx_bias_1, weights_1, ctx_maps_2, ctx_bias_2,
           weights_2):
    cm0T, cb0, w0T = _prep_layer(ctx_maps_0, ctx_bias_0, weights_0, 128, 1, 256)
    cm1T, cb1, w1T = _prep_layer(ctx_maps_1, ctx_bias_1, weights_1, 128, 1, 128)
    cm2T, cb2, w2T = _prep_layer(ctx_maps_2, ctx_bias_2, weights_2, 8, 0, 128)
    scalars = jnp.stack([base_bias, bias_0[0, 0, 0], bias_1[0, 0, 0]])

    rep = lambda i: (0, 0)
    blk = lambda i: (i, 0)
    grid = (_CB // _BB,)

    def chain(xc):
        a0, off0, off1, idx2 = pl.pallas_call(
            _tc_a_body,
            grid=grid,
            in_specs=[
                pl.BlockSpec((_BB, 256), blk),
                pl.BlockSpec(memory_space=pltpu.SMEM),
                pl.BlockSpec((256, 512), rep),
                pl.BlockSpec((1, 512), rep),
                pl.BlockSpec((256, 2048), rep),
                pl.BlockSpec((256, 512), rep),
                pl.BlockSpec((1, 512), rep),
                pl.BlockSpec((256, 32), rep),
                pl.BlockSpec((1, 32), rep),
            ],
            out_specs=[
                pl.BlockSpec((_BB, 2048), blk),
                pl.BlockSpec((_BB, 128), blk),
                pl.BlockSpec((_BB, 128), blk),
                pl.BlockSpec((_BB, 8), blk),
            ],
            out_shape=[
                jax.ShapeDtypeStruct((_CB, 2048), jnp.float32),
                jax.ShapeDtypeStruct((_CB, 128), jnp.int32),
                jax.ShapeDtypeStruct((_CB, 128), jnp.int32),
                jax.ShapeDtypeStruct((_CB, 8), jnp.float32),
            ],
        )(xc, scalars, cm0T, cb0, w0T, cm1T, cb1, cm2T, cb2)

        sel0 = _get_sc_select()(a0, off0)

        a1 = pl.pallas_call(
            functools.partial(_tc_b_body, bias_slot=1),
            grid=grid,
            in_specs=[
                pl.BlockSpec((_BB, 128), blk),
                pl.BlockSpec(memory_space=pltpu.SMEM),
                pl.BlockSpec((128, 2048), rep),
            ],
            out_specs=pl.BlockSpec((_BB, 2048), blk),
            out_shape=jax.ShapeDtypeStruct((_CB, 2048), jnp.float32),
        )(sel0, scalars, w1T)

        sel1 = _get_sc_select()(a1, off1)

        return pl.pallas_call(
            _tc_c_body,
            grid=grid,
            in_specs=[
                pl.BlockSpec((_BB, 128), blk),
                pl.BlockSpec(memory_space=pltpu.SMEM),
                pl.BlockSpec((128, 128), rep),
                pl.BlockSpec((_BB, 8), blk),
            ],
            out_specs=pl.BlockSpec((_BB, 1), blk),
            out_shape=jax.ShapeDtypeStruct((_CB, 1), jnp.float32),
        )(sel1, scalars, w2T, idx2)

    outs = [chain(x[c * _CB:(c + 1) * _CB]) for c in range(_NCHUNK)]
    return jnp.concatenate(outs, axis=0)


# speedup vs baseline: 1.1875x; 1.0949x over previous
import functools
import math

import jax
import jax.numpy as jnp
import numpy as np
from jax import lax
from jax.experimental import pallas as pl
from jax.experimental.pallas import tpu as pltpu
from jax.experimental.pallas import tpu_sc as plsc

_PRED_CLIP = 0.001
_LO = math.log(_PRED_CLIP / (1.0 - _PRED_CLIP))
_HI = math.log((1.0 - _PRED_CLIP) / _PRED_CLIP)
_BB = 256
_B = 1024
_NW = 32
_SPW = _B // _NW


def _pack_matrix(s):
    sd = ((4 * s + 127) // 128) * 128
    p = np.zeros((sd, 128), np.float32)
    for si in range(s):
        for i in range(4):
            p[si * 4 + i, si + 1] = float(1 << i)
    return p


_P0 = _pack_matrix(127)
_P1 = _pack_matrix(63)[:256]

_DNT = (((1,), (1,)), ((), ()))


def _dott(a, b):
    return lax.dot_general(a, b, _DNT, preferred_element_type=jnp.float32)


def _tc_a_body(x_ref, sc_ref, cm0_ref, cb0_ref, w0_ref, p0_ref, cm1_ref,
               cb1_ref, p1_ref, cm2_ref, cb2_ref, a0_ref, off0_ref,
               off1_ref, idx2_ref):
    x = x_ref[...]
    lane256 = lax.broadcasted_iota(jnp.int32, (1, 256), 1)
    lane128 = lax.broadcasted_iota(jnp.int32, (1, 128), 1).astype(jnp.float32)
    base = 16.0 * jnp.maximum(lane128 - 1.0, 0.0)

    xc = jnp.clip(x, _PRED_CLIP, 1.0 - _PRED_CLIP)
    l0 = jnp.log(xc / (1.0 - xc))
    l0 = jnp.where(lane256 == 0, sc_ref[0], l0)

    bits0 = (_dott(x, cm0_ref[...]) > cb0_ref[...]).astype(jnp.float32)
    off0_ref[...] = (jnp.dot(bits0, p0_ref[...],
                             preferred_element_type=jnp.float32)
                     + base).astype(jnp.int32)
    bits1 = (_dott(x, cm1_ref[...]) > cb1_ref[...]).astype(jnp.float32)
    off1 = (jnp.dot(bits1, p1_ref[...], preferred_element_type=jnp.float32)
            + base).astype(jnp.int32)
    off1_ref[...] = jnp.minimum(off1, 1023)
    bits2 = (_dott(x, cm2_ref[...]) > cb2_ref[...]).astype(jnp.float32)
    idx2_ref[...] = (bits2[:, 0:1] + 2.0 * bits2[:, 1:2]
                     + 4.0 * bits2[:, 2:3] + 8.0 * bits2[:, 3:4])
    a0_ref[...] = _dott(l0, w0_ref[...])


def _sc_sel_body(a_hbm, off_hbm, out_hbm, a_v, off_v, out_v):
    cid = lax.axis_index("c")
    sid = lax.axis_index("s")
    wid = sid * 2 + cid
    base = wid * _SPW
    pltpu.sync_copy(a_hbm.at[pl.ds(base, _SPW)], a_v)
    pltpu.sync_copy(off_hbm.at[pl.ds(base, _SPW)], off_v)

    def jbody(j, carry):
        jv = jnp.full((16,), j, jnp.int32)
        for g in range(8):
            off = off_v[j, pl.ds(g * 16, 16)]
            val = plsc.load_gather(a_v, [jv, off])
            out_v[j, pl.ds(g * 16, 16)] = val
        return carry

    lax.fori_loop(0, _SPW, jbody, 0)
    pltpu.sync_copy(out_v, out_hbm.at[pl.ds(base, _SPW)])


@functools.cache
def _get_sc_select(width):
    return pl.kernel(
        _sc_sel_body,
        out_type=jax.ShapeDtypeStruct((_B, 128), jnp.float32),
        mesh=plsc.VectorSubcoreMesh(core_axis_name="c", subcore_axis_name="s"),
        compiler_params=pltpu.CompilerParams(needs_layout_passes=False),
        scratch_types=[
            pltpu.VMEM((_SPW, width), jnp.float32),
            pltpu.VMEM((_SPW, 128), jnp.int32),
            pltpu.VMEM((_SPW, 128), jnp.float32),
        ],
    )


def _tc_b_body(sel_ref, sc_ref, w_ref, a_ref):
    lane128 = lax.broadcasted_iota(jnp.int32, (1, 128), 1)
    l = jnp.where(lane128 == 0, sc_ref[1],
                  jnp.clip(sel_ref[...], _LO, _HI))
    a_ref[...] = _dott(l, w_ref[...])


def _tc_c_body(sel_ref, sc_ref, w2_ref, idx2_ref, o_ref):
    lane128 = lax.broadcasted_iota(jnp.int32, (1, 128), 1)
    lane16 = lax.broadcasted_iota(jnp.int32, (1, 16), 1).astype(jnp.float32)
    l2 = jnp.where(lane128 == 0, sc_ref[2],
                   jnp.clip(sel_ref[...], _LO, _HI))
    a2 = _dott(l2, w2_ref[...])
    out2 = jnp.sum(jnp.where(lane16 == idx2_ref[...], a2, 0.0),
                   axis=1, keepdims=True)
    o_ref[...] = jax.nn.sigmoid(jnp.clip(out2, _LO, _HI))


def kernel(x, base_bias, bias_0, bias_1, ctx_maps_0, ctx_bias_0, weights_0,
           ctx_maps_1, ctx_bias_1, weights_1, ctx_maps_2, ctx_bias_2,
           weights_2):
    cm0 = jnp.pad(ctx_maps_0.reshape(508, 256), ((0, 4), (0, 0)))
    cb0 = jnp.pad(ctx_bias_0.reshape(1, 508), ((0, 0), (0, 4)),
                  constant_values=jnp.inf)
    w0 = jnp.pad(weights_0.reshape(2032, 256), ((0, 16), (0, 0)))
    cm1 = jnp.pad(ctx_maps_1.reshape(252, 256), ((0, 4), (0, 0)))
    cb1 = jnp.pad(ctx_bias_1.reshape(1, 252), ((0, 0), (0, 4)),
                  constant_values=jnp.inf)
    w1 = jnp.pad(weights_1.reshape(1008, 128), ((0, 16), (0, 0)))
    cm2 = jnp.pad(ctx_maps_2.reshape(4, 256), ((0, 4), (0, 0)))
    cb2 = jnp.pad(ctx_bias_2.reshape(1, 4), ((0, 0), (0, 4)),
                  constant_values=jnp.inf)
    w2 = jnp.pad(weights_2.reshape(16, 64), ((0, 0), (0, 64)))
    scalars = jnp.stack([base_bias, bias_0[0, 0, 0], bias_1[0, 0, 0]])
    p0 = jnp.asarray(_P0)
    p1 = jnp.asarray(_P1)

    rep = lambda i: (0, 0)
    blk = lambda i: (i, 0)
    grid = (_B // _BB,)

    a0, off0, off1, idx2 = pl.pallas_call(
        _tc_a_body,
        grid=grid,
        in_specs=[
            pl.BlockSpec((_BB, 256), blk),
            pl.BlockSpec(memory_space=pltpu.SMEM),
            pl.BlockSpec((512, 256), rep),
            pl.BlockSpec((1, 512), rep),
            pl.BlockSpec((2048, 256), rep),
            pl.BlockSpec((512, 128), rep),
            pl.BlockSpec((256, 256), rep),
            pl.BlockSpec((1, 256), rep),
            pl.BlockSpec((256, 128), rep),
            pl.BlockSpec((8, 256), rep),
            pl.BlockSpec((1, 8), rep),
        ],
        out_specs=[
            pl.BlockSpec((_BB, 2048), blk),
            pl.BlockSpec((_BB, 128), blk),
            pl.BlockSpec((_BB, 128), blk),
            pl.BlockSpec((_BB, 1), blk),
        ],
        out_shape=[
            jax.ShapeDtypeStruct((_B, 2048), jnp.float32),
            jax.ShapeDtypeStruct((_B, 128), jnp.int32),
            jax.ShapeDtypeStruct((_B, 128), jnp.int32),
            jax.ShapeDtypeStruct((_B, 1), jnp.float32),
        ],
    )(x, scalars, cm0, cb0, w0, p0, cm1, cb1, p1, cm2, cb2)

    sel0 = _get_sc_select(2048)(a0, off0)

    a1 = pl.pallas_call(
        _tc_b_body,
        grid=grid,
        in_specs=[
            pl.BlockSpec((_BB, 128), blk),
            pl.BlockSpec(memory_space=pltpu.SMEM),
            pl.BlockSpec((1024, 128), rep),
        ],
        out_specs=pl.BlockSpec((_BB, 1024), blk),
        out_shape=jax.ShapeDtypeStruct((_B, 1024), jnp.float32),
    )(sel0, scalars, w1)

    sel1 = _get_sc_select(1024)(a1, off1)

    probs = pl.pallas_call(
        _tc_c_body,
        grid=grid,
        in_specs=[
            pl.BlockSpec((_BB, 128), blk),
            pl.BlockSpec(memory_space=pltpu.SMEM),
            pl.BlockSpec((16, 128), rep),
            pl.BlockSpec((_BB, 1), blk),
        ],
        out_specs=pl.BlockSpec((_BB, 1), blk),
        out_shape=jax.ShapeDtypeStruct((_B, 1), jnp.float32),
    )(sel1, scalars, w2, idx2)
    return probs


# speedup vs baseline: 1.2380x; 1.0425x over previous
import functools
import math

import jax
import jax.numpy as jnp
import numpy as np
from jax import lax
from jax.experimental import pallas as pl
from jax.experimental.pallas import tpu as pltpu
from jax.experimental.pallas import tpu_sc as plsc

_PRED_CLIP = 0.001
_LO = math.log(_PRED_CLIP / (1.0 - _PRED_CLIP))
_HI = math.log((1.0 - _PRED_CLIP) / _PRED_CLIP)
_BB = 256
_B = 1024
_NW = 32
_SPW = _B // _NW


def _pack_matrix(s):
    p = np.zeros((4 * s, 128), np.float32)
    for si in range(s):
        for i in range(4):
            p[si * 4 + i, si + 1] = float(1 << i)
    return p


_P0 = _pack_matrix(127)
_P1 = _pack_matrix(63)

_DNT = (((1,), (1,)), ((), ()))


def _dott(a, b):
    return lax.dot_general(a, b, _DNT, preferred_element_type=jnp.float32)


def _tc_a_body(x_ref, sc_ref, cm0_ref, cb0_ref, w0_ref, p0_ref, cm1_ref,
               cb1_ref, p1_ref, cm2_ref, cb2_ref, a0_ref, off0_ref,
               off1_ref, idx2_ref):
    x = x_ref[...]
    lane256 = lax.broadcasted_iota(jnp.int32, (1, 256), 1)
    lane128 = lax.broadcasted_iota(jnp.int32, (1, 128), 1).astype(jnp.float32)
    base = 16.0 * jnp.maximum(lane128 - 1.0, 0.0)

    xc = jnp.clip(x, _PRED_CLIP, 1.0 - _PRED_CLIP)
    l0 = jnp.log(xc / (1.0 - xc))
    l0 = jnp.where(lane256 == 0, sc_ref[0], l0)

    bits0 = (_dott(x, cm0_ref[...]) > cb0_ref[...]).astype(jnp.float32)
    off0_ref[...] = (jnp.dot(bits0, p0_ref[...],
                             preferred_element_type=jnp.float32)
                     + base).astype(jnp.int32)
    bits1 = (_dott(x, cm1_ref[...]) > cb1_ref[...]).astype(jnp.float32)
    off1 = (jnp.dot(bits1, p1_ref[...], preferred_element_type=jnp.float32)
            + base).astype(jnp.int32)
    off1_ref[...] = jnp.minimum(off1, 1007)
    bits2 = (_dott(x, cm2_ref[...]) > cb2_ref[...]).astype(jnp.float32)
    idx2_ref[...] = (bits2[:, 0:1] + 2.0 * bits2[:, 1:2]
                     + 4.0 * bits2[:, 2:3] + 8.0 * bits2[:, 3:4])
    a0_ref[...] = _dott(l0, w0_ref[...])


def _sc_sel_body(a_hbm, off_hbm, out_hbm, a_v, off_v, out_v):
    cid = lax.axis_index("c")
    sid = lax.axis_index("s")
    wid = sid * 2 + cid
    base = wid * _SPW
    pltpu.sync_copy(a_hbm.at[pl.ds(base, _SPW)], a_v)
    pltpu.sync_copy(off_hbm.at[pl.ds(base, _SPW)], off_v)

    def jbody(j, carry):
        jv = jnp.full((16,), j, jnp.int32)
        for g in range(8):
            off = off_v[j, pl.ds(g * 16, 16)]
            val = plsc.load_gather(a_v, [jv, off])
            out_v[j, pl.ds(g * 16, 16)] = val
        return carry

    lax.fori_loop(0, _SPW, jbody, 0)
    pltpu.sync_copy(out_v, out_hbm.at[pl.ds(base, _SPW)])


@functools.cache
def _get_sc_select(width):
    return pl.kernel(
        _sc_sel_body,
        out_type=jax.ShapeDtypeStruct((_B, 128), jnp.float32),
        mesh=plsc.VectorSubcoreMesh(core_axis_name="c", subcore_axis_name="s"),
        compiler_params=pltpu.CompilerParams(needs_layout_passes=False),
        scratch_types=[
            pltpu.VMEM((_SPW, width), jnp.float32),
            pltpu.VMEM((_SPW, 128), jnp.int32),
            pltpu.VMEM((_SPW, 128), jnp.float32),
        ],
    )


def _tc_b_body(sel_ref, sc_ref, w_ref, a_ref):
    lane128 = lax.broadcasted_iota(jnp.int32, (1, 128), 1)
    l = jnp.where(lane128 == 0, sc_ref[1],
                  jnp.clip(sel_ref[...], _LO, _HI))
    a_ref[...] = _dott(l, w_ref[...])


def _tc_c_body(sel_ref, sc_ref, w2_ref, idx2_ref, o_ref):
    lane64 = lax.broadcasted_iota(jnp.int32, (1, 64), 1)
    lane16 = lax.broadcasted_iota(jnp.int32, (1, 16), 1).astype(jnp.float32)
    l2 = jnp.where(lane64 == 0, sc_ref[2],
                   jnp.clip(sel_ref[:, 0:64], _LO, _HI))
    a2 = _dott(l2, w2_ref[...])
    out2 = jnp.sum(jnp.where(lane16 == idx2_ref[...], a2, 0.0),
                   axis=1, keepdims=True)
    o_ref[...] = jax.nn.sigmoid(jnp.clip(out2, _LO, _HI))


def kernel(x, base_bias, bias_0, bias_1, ctx_maps_0, ctx_bias_0, weights_0,
           ctx_maps_1, ctx_bias_1, weights_1, ctx_maps_2, ctx_bias_2,
           weights_2):
    cm0 = ctx_maps_0.reshape(508, 256)
    cb0 = ctx_bias_0.reshape(1, 508)
    w0 = weights_0.reshape(2032, 256)
    cm1 = ctx_maps_1.reshape(252, 256)
    cb1 = ctx_bias_1.reshape(1, 252)
    w1 = weights_1.reshape(1008, 128)
    cm2 = ctx_maps_2.reshape(4, 256)
    cb2 = ctx_bias_2.reshape(1, 4)
    w2 = weights_2.reshape(16, 64)
    scalars = jnp.stack([base_bias, bias_0[0, 0, 0], bias_1[0, 0, 0]])
    p0 = jnp.asarray(_P0)
    p1 = jnp.asarray(_P1)

    rep = lambda i: (0, 0)
    blk = lambda i: (i, 0)
    grid = (_B // _BB,)

    a0, off0, off1, idx2 = pl.pallas_call(
        _tc_a_body,
        grid=grid,
        in_specs=[
            pl.BlockSpec((_BB, 256), blk),
            pl.BlockSpec(memory_space=pltpu.SMEM),
            pl.BlockSpec((508, 256), rep),
            pl.BlockSpec((1, 508), rep),
            pl.BlockSpec((2032, 256), rep),
            pl.BlockSpec((508, 128), rep),
            pl.BlockSpec((252, 256), rep),
            pl.BlockSpec((1, 252), rep),
            pl.BlockSpec((252, 128), rep),
            pl.BlockSpec((4, 256), rep),
            pl.BlockSpec((1, 4), rep),
        ],
        out_specs=[
            pl.BlockSpec((_BB, 2032), blk),
            pl.BlockSpec((_BB, 128), blk),
            pl.BlockSpec((_BB, 128), blk),
            pl.BlockSpec((_BB, 1), blk),
        ],
        out_shape=[
            jax.ShapeDtypeStruct((_B, 2032), jnp.float32),
            jax.ShapeDtypeStruct((_B, 128), jnp.int32),
            jax.ShapeDtypeStruct((_B, 128), jnp.int32),
            jax.ShapeDtypeStruct((_B, 1), jnp.float32),
        ],
    )(x, scalars, cm0, cb0, w0, p0, cm1, cb1, p1, cm2, cb2)

    sel0 = _get_sc_select(2032)(a0, off0)

    a1 = pl.pallas_call(
        _tc_b_body,
        grid=grid,
        in_specs=[
            pl.BlockSpec((_BB, 128), blk),
            pl.BlockSpec(memory_space=pltpu.SMEM),
            pl.BlockSpec((1008, 128), rep),
        ],
        out_specs=pl.BlockSpec((_BB, 1008), blk),
        out_shape=jax.ShapeDtypeStruct((_B, 1008), jnp.float32),
    )(sel0, scalars, w1)

    sel1 = _get_sc_select(1008)(a1, off1)

    probs = pl.pallas_call(
        _tc_c_body,
        grid=grid,
        in_specs=[
            pl.BlockSpec((_BB, 128), blk),
            pl.BlockSpec(memory_space=pltpu.SMEM),
            pl.BlockSpec((16, 64), rep),
            pl.BlockSpec((_BB, 1), blk),
        ],
        out_specs=pl.BlockSpec((_BB, 1), blk),
        out_shape=jax.ShapeDtypeStruct((_B, 1), jnp.float32),
    )(sel1, scalars, w2, idx2)
    return probs


# speedup vs baseline: 1.3287x; 1.0733x over previous
import functools
import math

import jax
import jax.numpy as jnp
import numpy as np
from jax import lax
from jax.experimental import pallas as pl
from jax.experimental.pallas import tpu as pltpu
from jax.experimental.pallas import tpu_sc as plsc

_PRED_CLIP = 0.001
_LO = math.log(_PRED_CLIP / (1.0 - _PRED_CLIP))
_HI = math.log((1.0 - _PRED_CLIP) / _PRED_CLIP)
_BB = 256
_B = 1024
_NW = 32
_SPW = _B // _NW


def _pack_matrix(s):
    p = np.zeros((4 * s, 128), np.float32)
    for si in range(s):
        for i in range(4):
            p[si * 4 + i, si + 1] = float(1 << i)
    return p


_P0 = _pack_matrix(127)
_P1 = _pack_matrix(63)

_DNT = (((1,), (1,)), ((), ()))


def _dott(a, b):
    return lax.dot_general(a, b, _DNT, preferred_element_type=jnp.float32)


def _tc_a_body(x_ref, sc_ref, cm0_ref, cb0_ref, w0_ref, p0_ref, cm1_ref,
               cb1_ref, p1_ref, cm2_ref, cb2_ref, a0_ref, off0_ref,
               off1_ref, idx2_ref):
    x = x_ref[...]
    cm0 = cm0_ref[...].reshape(508, 256)
    w0 = w0_ref[...].reshape(2032, 256)
    cm1 = cm1_ref[...].reshape(252, 256)
    cm2 = cm2_ref[...].reshape(4, 256)
    lane256 = lax.broadcasted_iota(jnp.int32, (1, 256), 1)
    lane128 = lax.broadcasted_iota(jnp.int32, (1, 128), 1).astype(jnp.float32)
    base = 16.0 * jnp.maximum(lane128 - 1.0, 0.0)

    xc = jnp.clip(x, _PRED_CLIP, 1.0 - _PRED_CLIP)
    l0 = jnp.log(xc / (1.0 - xc))
    l0 = jnp.where(lane256 == 0, sc_ref[0], l0)

    bits0 = (_dott(x, cm0) > cb0_ref[...]).astype(jnp.float32)
    off0_ref[...] = (jnp.dot(bits0, p0_ref[...],
                             preferred_element_type=jnp.float32)
                     + base).astype(jnp.int32)
    bits1 = (_dott(x, cm1) > cb1_ref[...]).astype(jnp.float32)
    off1 = (jnp.dot(bits1, p1_ref[...], preferred_element_type=jnp.float32)
            + base).astype(jnp.int32)
    off1_ref[...] = jnp.minimum(off1, 1007)
    bits2 = (_dott(x, cm2) > cb2_ref[...]).astype(jnp.float32)
    idx2_ref[...] = (bits2[:, 0:1] + 2.0 * bits2[:, 1:2]
                     + 4.0 * bits2[:, 2:3] + 8.0 * bits2[:, 3:4])
    a0_ref[...] = _dott(l0, w0)


def _sc_sel_body(a_hbm, off_hbm, out_hbm, a_v, off_v, out_v):
    cid = lax.axis_index("c")
    sid = lax.axis_index("s")
    wid = sid * 2 + cid
    base = wid * _SPW
    pltpu.sync_copy(a_hbm.at[pl.ds(base, _SPW)], a_v)
    pltpu.sync_copy(off_hbm.at[pl.ds(base, _SPW)], off_v)

    def jbody(j, carry):
        jv = jnp.full((16,), j, jnp.int32)
        for g in range(8):
            off = off_v[j, pl.ds(g * 16, 16)]
            val = plsc.load_gather(a_v, [jv, off])
            out_v[j, pl.ds(g * 16, 16)] = val
        return carry

    lax.fori_loop(0, _SPW, jbody, 0)
    pltpu.sync_copy(out_v, out_hbm.at[pl.ds(base, _SPW)])


@functools.cache
def _get_sc_select(width):
    return pl.kernel(
        _sc_sel_body,
        out_type=jax.ShapeDtypeStruct((_B, 128), jnp.float32),
        mesh=plsc.VectorSubcoreMesh(core_axis_name="c", subcore_axis_name="s"),
        compiler_params=pltpu.CompilerParams(needs_layout_passes=False),
        scratch_types=[
            pltpu.VMEM((_SPW, width), jnp.float32),
            pltpu.VMEM((_SPW, 128), jnp.int32),
            pltpu.VMEM((_SPW, 128), jnp.float32),
        ],
    )


def _tc_b_body(sel_ref, sc_ref, w_ref, a_ref):
    lane128 = lax.broadcasted_iota(jnp.int32, (1, 128), 1)
    l = jnp.where(lane128 == 0, sc_ref[1],
                  jnp.clip(sel_ref[...], _LO, _HI))
    a_ref[...] = _dott(l, w_ref[...].reshape(1008, 128))


def _tc_c_body(sel_ref, sc_ref, w2_ref, idx2_ref, o_ref):
    lane64 = lax.broadcasted_iota(jnp.int32, (1, 64), 1)
    lane16 = lax.broadcasted_iota(jnp.int32, (1, 16), 1).astype(jnp.float32)
    l2 = jnp.where(lane64 == 0, sc_ref[2],
                   jnp.clip(sel_ref[:, 0:64], _LO, _HI))
    a2 = _dott(l2, w2_ref[...])
    out2 = jnp.sum(jnp.where(lane16 == idx2_ref[...], a2, 0.0),
                   axis=1, keepdims=True)
    o_ref[...] = jax.nn.sigmoid(jnp.clip(out2, _LO, _HI))


def kernel(x, base_bias, bias_0, bias_1, ctx_maps_0, ctx_bias_0, weights_0,
           ctx_maps_1, ctx_bias_1, weights_1, ctx_maps_2, ctx_bias_2,
           weights_2):
    cb0 = ctx_bias_0.reshape(1, 508)
    cb1 = ctx_bias_1.reshape(1, 252)
    cb2 = ctx_bias_2.reshape(1, 4)
    w2 = weights_2.reshape(16, 64)
    scalars = jnp.stack([base_bias, bias_0[0, 0, 0], bias_1[0, 0, 0]])
    p0 = jnp.asarray(_P0)
    p1 = jnp.asarray(_P1)

    rep = lambda i: (0, 0)
    rep4 = lambda i: (0, 0, 0, 0)
    blk = lambda i: (i, 0)
    grid = (_B // _BB,)
    grid_a = (1,)

    a0, off0, off1, idx2 = pl.pallas_call(
        _tc_a_body,
        grid=grid_a,
        in_specs=[
            pl.BlockSpec((_B, 256), blk),
            pl.BlockSpec(memory_space=pltpu.SMEM),
            pl.BlockSpec((1, 127, 4, 256), rep4),
            pl.BlockSpec((1, 508), rep),
            pl.BlockSpec((1, 127, 16, 256), rep4),
            pl.BlockSpec((508, 128), rep),
            pl.BlockSpec((1, 63, 4, 256), rep4),
            pl.BlockSpec((1, 252), rep),
            pl.BlockSpec((252, 128), rep),
            pl.BlockSpec((1, 1, 4, 256), rep4),
            pl.BlockSpec((1, 4), rep),
        ],
        out_specs=[
            pl.BlockSpec((_B, 2032), blk),
            pl.BlockSpec((_B, 128), blk),
            pl.BlockSpec((_B, 128), blk),
            pl.BlockSpec((_B, 1), blk),
        ],
        out_shape=[
            jax.ShapeDtypeStruct((_B, 2032), jnp.float32),
            jax.ShapeDtypeStruct((_B, 128), jnp.int32),
            jax.ShapeDtypeStruct((_B, 128), jnp.int32),
            jax.ShapeDtypeStruct((_B, 1), jnp.float32),
        ],
    )(x, scalars, ctx_maps_0, cb0, weights_0, p0, ctx_maps_1, cb1, p1,
      ctx_maps_2, cb2)

    sel0 = _get_sc_select(2032)(a0, off0)

    a1 = pl.pallas_call(
        _tc_b_body,
        grid=grid,
        in_specs=[
            pl.BlockSpec((_BB, 128), blk),
            pl.BlockSpec(memory_space=pltpu.SMEM),
            pl.BlockSpec((1, 63, 16, 128), rep4),
        ],
        out_specs=pl.BlockSpec((_BB, 1008), blk),
        out_shape=jax.ShapeDtypeStruct((_B, 1008), jnp.float32),
    )(sel0, scalars, weights_1)

    sel1 = _get_sc_select(1008)(a1, off1)

    probs = pl.pallas_call(
        _tc_c_body,
        grid=grid,
        in_specs=[
            pl.BlockSpec((_BB, 128), blk),
            pl.BlockSpec(memory_space=pltpu.SMEM),
            pl.BlockSpec((16, 64), rep),
            pl.BlockSpec((_BB, 1), blk),
        ],
        out_specs=pl.BlockSpec((_BB, 1), blk),
        out_shape=jax.ShapeDtypeStruct((_B, 1), jnp.float32),
    )(sel1, scalars, w2, idx2)
    return probs


# speedup vs baseline: 1.3442x; 1.0116x over previous
import functools
import math

import jax
import jax.numpy as jnp
import numpy as np
from jax import lax
from jax.experimental import pallas as pl
from jax.experimental.pallas import tpu as pltpu
from jax.experimental.pallas import tpu_sc as plsc

_PRED_CLIP = 0.001
_LO = math.log(_PRED_CLIP / (1.0 - _PRED_CLIP))
_HI = math.log((1.0 - _PRED_CLIP) / _PRED_CLIP)
_BB = 256
_B = 1024
_NW = 32
_SPW = _B // _NW


def _pack_matrix(s):
    p = np.zeros((4 * s, 128), np.float32)
    for si in range(s):
        for i in range(4):
            p[si * 4 + i, si + 1] = float(1 << i)
    return p


_P0 = _pack_matrix(127)
_P1 = _pack_matrix(63)

_DNT = (((1,), (1,)), ((), ()))


def _dott(a, b):
    return lax.dot_general(a, b, _DNT, preferred_element_type=jnp.float32)


def _tc_a_body(x_ref, sc_ref, cm0_ref, cb0_ref, w0_ref, p0_ref, cm1_ref,
               cb1_ref, p1_ref, cm2_ref, cb2_ref, a0_ref, off0_ref,
               off1_ref, idx2_ref):
    x = x_ref[...]
    cm0 = cm0_ref[...].reshape(508, 256)
    w0 = w0_ref[...].reshape(2032, 256)
    cm1 = cm1_ref[...].reshape(252, 256)
    cm2 = cm2_ref[...].reshape(4, 256)
    lane256 = lax.broadcasted_iota(jnp.int32, (1, 256), 1)
    lane128 = lax.broadcasted_iota(jnp.int32, (1, 128), 1).astype(jnp.float32)
    base = 16.0 * jnp.maximum(lane128 - 1.0, 0.0)

    xc = jnp.clip(x, _PRED_CLIP, 1.0 - _PRED_CLIP)
    l0 = jnp.log(xc / (1.0 - xc))
    l0 = jnp.where(lane256 == 0, sc_ref[0], l0)

    bits0 = (_dott(x, cm0) > cb0_ref[...]).astype(jnp.float32)
    off0_ref[...] = (jnp.dot(bits0, p0_ref[...],
                             preferred_element_type=jnp.float32)
                     + base).astype(jnp.int32)
    bits1 = (_dott(x, cm1) > cb1_ref[...]).astype(jnp.float32)
    off1 = (jnp.dot(bits1, p1_ref[...], preferred_element_type=jnp.float32)
            + base).astype(jnp.int32)
    off1_ref[...] = jnp.minimum(off1, 1007)
    bits2 = (_dott(x, cm2) > cb2_ref[...]).astype(jnp.float32)
    idx2_ref[...] = (bits2[:, 0:1] + 2.0 * bits2[:, 1:2]
                     + 4.0 * bits2[:, 2:3] + 8.0 * bits2[:, 3:4])
    a0_ref[...] = _dott(l0, w0)


_NCK = 4
_CKR = _SPW // _NCK


def _sc_sel_body(a_hbm, off_hbm, out_hbm, a_v, off_v, out_v, *sems):
    cid = lax.axis_index("c")
    sid = lax.axis_index("s")
    wid = sid * 2 + cid
    base = wid * _SPW
    copies = [
        pltpu.async_copy(a_hbm.at[pl.ds(base + c * _CKR, _CKR)],
                         a_v.at[pl.ds(c * _CKR, _CKR)], sems[c])
        for c in range(_NCK)
    ]
    pltpu.sync_copy(off_hbm.at[pl.ds(base, _SPW)], off_v)

    def jbody(j, carry):
        jv = jnp.full((16,), j, jnp.int32)
        for g in range(8):
            off = off_v[j, pl.ds(g * 16, 16)]
            val = plsc.load_gather(a_v, [jv, off])
            out_v[j, pl.ds(g * 16, 16)] = val
        return carry

    for c in range(_NCK):
        copies[c].wait()
        lax.fori_loop(c * _CKR, (c + 1) * _CKR, jbody, 0)
    pltpu.sync_copy(out_v, out_hbm.at[pl.ds(base, _SPW)])


@functools.cache
def _get_sc_select(width):
    return pl.kernel(
        _sc_sel_body,
        out_type=jax.ShapeDtypeStruct((_B, 128), jnp.float32),
        mesh=plsc.VectorSubcoreMesh(core_axis_name="c", subcore_axis_name="s"),
        compiler_params=pltpu.CompilerParams(needs_layout_passes=False),
        scratch_types=[
            pltpu.VMEM((_SPW, width), jnp.float32),
            pltpu.VMEM((_SPW, 128), jnp.int32),
            pltpu.VMEM((_SPW, 128), jnp.float32),
        ] + [pltpu.SemaphoreType.DMA] * _NCK,
    )


def _tc_b_body(sel_ref, sc_ref, w_ref, a_ref):
    lane128 = lax.broadcasted_iota(jnp.int32, (1, 128), 1)
    l = jnp.where(lane128 == 0, sc_ref[1],
                  jnp.clip(sel_ref[...], _LO, _HI))
    a_ref[...] = _dott(l, w_ref[...].reshape(1008, 128))


def _tc_c_body(sel_ref, sc_ref, w2_ref, idx2_ref, o_ref):
    lane64 = lax.broadcasted_iota(jnp.int32, (1, 64), 1)
    lane16 = lax.broadcasted_iota(jnp.int32, (1, 16), 1).astype(jnp.float32)
    l2 = jnp.where(lane64 == 0, sc_ref[2],
                   jnp.clip(sel_ref[:, 0:64], _LO, _HI))
    a2 = _dott(l2, w2_ref[...])
    out2 = jnp.sum(jnp.where(lane16 == idx2_ref[...], a2, 0.0),
                   axis=1, keepdims=True)
    o_ref[...] = jax.nn.sigmoid(jnp.clip(out2, _LO, _HI))


def kernel(x, base_bias, bias_0, bias_1, ctx_maps_0, ctx_bias_0, weights_0,
           ctx_maps_1, ctx_bias_1, weights_1, ctx_maps_2, ctx_bias_2,
           weights_2):
    cb0 = ctx_bias_0.reshape(1, 508)
    cb1 = ctx_bias_1.reshape(1, 252)
    cb2 = ctx_bias_2.reshape(1, 4)
    w2 = weights_2.reshape(16, 64)
    scalars = jnp.stack([base_bias, bias_0[0, 0, 0], bias_1[0, 0, 0]])
    p0 = jnp.asarray(_P0)
    p1 = jnp.asarray(_P1)

    rep = lambda i: (0, 0)
    rep4 = lambda i: (0, 0, 0, 0)
    blk = lambda i: (i, 0)
    grid = (_B // _BB,)
    grid_a = (1,)

    a0, off0, off1, idx2 = pl.pallas_call(
        _tc_a_body,
        grid=grid_a,
        in_specs=[
            pl.BlockSpec((_B, 256), blk),
            pl.BlockSpec(memory_space=pltpu.SMEM),
            pl.BlockSpec((1, 127, 4, 256), rep4),
            pl.BlockSpec((1, 508), rep),
            pl.BlockSpec((1, 127, 16, 256), rep4),
            pl.BlockSpec((508, 128), rep),
            pl.BlockSpec((1, 63, 4, 256), rep4),
            pl.BlockSpec((1, 252), rep),
            pl.BlockSpec((252, 128), rep),
            pl.BlockSpec((1, 1, 4, 256), rep4),
            pl.BlockSpec((1, 4), rep),
        ],
        out_specs=[
            pl.BlockSpec((_B, 2032), blk),
            pl.BlockSpec((_B, 128), blk),
            pl.BlockSpec((_B, 128), blk),
            pl.BlockSpec((_B, 1), blk),
        ],
        out_shape=[
            jax.ShapeDtypeStruct((_B, 2032), jnp.float32),
            jax.ShapeDtypeStruct((_B, 128), jnp.int32),
            jax.ShapeDtypeStruct((_B, 128), jnp.int32),
            jax.ShapeDtypeStruct((_B, 1), jnp.float32),
        ],
    )(x, scalars, ctx_maps_0, cb0, weights_0, p0, ctx_maps_1, cb1, p1,
      ctx_maps_2, cb2)

    sel0 = _get_sc_select(2032)(a0, off0)

    a1 = pl.pallas_call(
        _tc_b_body,
        grid=grid,
        in_specs=[
            pl.BlockSpec((_BB, 128), blk),
            pl.BlockSpec(memory_space=pltpu.SMEM),
            pl.BlockSpec((1, 63, 16, 128), rep4),
        ],
        out_specs=pl.BlockSpec((_BB, 1008), blk),
        out_shape=jax.ShapeDtypeStruct((_B, 1008), jnp.float32),
    )(sel0, scalars, weights_1)

    sel1 = _get_sc_select(1008)(a1, off1)

    probs = pl.pallas_call(
        _tc_c_body,
        grid=grid,
        in_specs=[
            pl.BlockSpec((_BB, 128), blk),
            pl.BlockSpec(memory_space=pltpu.SMEM),
            pl.BlockSpec((16, 64), rep),
            pl.BlockSpec((_BB, 1), blk),
        ],
        out_specs=pl.BlockSpec((_BB, 1), blk),
        out_shape=jax.ShapeDtypeStruct((_B, 1), jnp.float32),
    )(sel1, scalars, w2, idx2)
    return probs
